# baseline (speedup 1.0000x reference)
import functools

import jax
import jax.numpy as jnp
from jax import lax
from jax.experimental import pallas as pl
from jax.experimental.pallas import tpu as pltpu
from jax.experimental.pallas import tpu_sc as plsc

_C = 128
_LANES = 16


_HALF = 512000


def _tc_repack(tab_t):
    D, V = tab_t.shape
    K = 4096
    grid = _HALF // K
    nb = _HALF // K
    last = (V + K - 1) // K - 1

    def body(lo_ref, hi_ref, o_ref):
        o_ref[:, 0:D] = jnp.transpose(lo_ref[...], (1, 0))
        o_ref[:, D:2 * D] = jnp.transpose(hi_ref[...], (1, 0))

    return pl.pallas_call(
        body,
        grid=(grid,),
        in_specs=[
            pl.BlockSpec((D, K), lambda i: (0, i)),
            pl.BlockSpec((D, K), lambda i: (0, jnp.minimum(i + nb, last))),
        ],
        out_specs=pl.BlockSpec((K, 2 * D), lambda i: (i, 0)),
        out_shape=jax.ShapeDtypeStruct((_HALF, 2 * D), jnp.float32),
    )(tab_t, tab_t)


def _sc_gather_transposed(table, idx2, H, B):
    D = table.shape[1]
    n_chunks = idx2.shape[0]
    CB = B // _C
    info = plsc.get_sparse_core_info()
    num_cores = info.num_cores
    NW = info.num_cores * info.num_subcores
    per_w = n_chunks // NW
    mesh = plsc.VectorSubcoreMesh(core_axis_name="c", subcore_axis_name="s")

    @functools.partial(
        pl.kernel,
        mesh=mesh,
        out_type=jax.ShapeDtypeStruct((H, D // 8, CB, 8, _C), jnp.float32),
        scratch_types=[
            pltpu.VMEM((per_w, _C), jnp.int32),
            pltpu.VMEM((_C, D), jnp.float32),
            pltpu.VMEM((_C, D), jnp.float32),
            pltpu.VMEM((D, _C), jnp.float32),
            pltpu.VMEM((D, _C), jnp.float32),
            pltpu.SemaphoreType.DMA,
            pltpu.SemaphoreType.DMA,
            pltpu.SemaphoreType.DMA,
        ],
        compiler_params=pltpu.CompilerParams(use_tc_tiling_on_sc=False,
                                             needs_layout_passes=False),
    )
    def k(table_hbm, idx_hbm, out_hbm, idx_v, rows0, rows1, tr0, tr1,
          sem0, sem1, osem):
        wid = lax.axis_index("s") * num_cores + lax.axis_index("c")
        k0 = wid * per_w
        pltpu.sync_copy(idx_hbm.at[pl.ds(k0, per_w), :], idx_v)
        iota = lax.iota(jnp.int32, _LANES)
        iotas = [iota + b0 for b0 in range(0, _C, _LANES)]

        def transpose(rows, tr):
            for d in range(D):
                col = jnp.full((_LANES,), d, dtype=jnp.int32)
                for i in range(_C // _LANES):
                    vals = plsc.load_gather(rows, [iotas[i], col])
                    tr[d, pl.ds(i * _LANES, _LANES)] = vals

        def write_out(tr, k_abs):
            h = k_abs // CB
            c = lax.rem(k_abs, CB)
            return [pltpu.async_copy(tr.at[pl.ds(8 * r, 8), :],
                                     out_hbm.at[h, r, c], osem)
                    for r in range(D // 8)]

        def body(t, carry):
            a = 2 * t
            g0 = pltpu.async_copy(table_hbm.at[idx_v.at[a]], rows0, sem0)
            g1 = pltpu.async_copy(table_hbm.at[idx_v.at[a + 1]], rows1, sem1)
            g0.wait()
            transpose(rows0, tr0)
            w0 = write_out(tr0, k0 + a)
            g1.wait()
            transpose(rows1, tr1)
            for w in w0:
                w.wait()
            w1 = write_out(tr1, k0 + a + 1)
            for w in w1:
                w.wait()
            return carry

        lax.fori_loop(0, per_w // 2, body, 0)

    return k(table, idx2)


def _tc_times_t(row1_t, tp_t, fc_w, fc_b):
    H, B = row1_t.shape
    D = fc_w.shape[1]
    HB = 8

    def body(r1_ref, tp_ref, w_ref, b_ref, o_ref):
        t = jnp.log(tp_ref[...] - r1_ref[...] + 1.0)
        w = jnp.reshape(w_ref[...], (1, D, 1))
        bb = jnp.reshape(b_ref[...], (1, D, 1))
        o_ref[...] = t[:, None, :] * w + bb

    return pl.pallas_call(
        body,
        grid=(H // HB,),
        in_specs=[
            pl.BlockSpec((HB, B), lambda i: (i, 0)),
            pl.BlockSpec((1, B), lambda i: (0, 0)),
            pl.BlockSpec((1, D), lambda i: (0, 0)),
            pl.BlockSpec((1, D), lambda i: (0, 0)),
        ],
        out_specs=pl.BlockSpec((HB, D, B), lambda i: (i, 0, 0)),
        out_shape=jax.ShapeDtypeStruct((H, D, B), jnp.float32),
    )(row1_t, tp_t, fc_w, fc_b)


def kernel(row0, row1, row2, row3, row4, row5, row6, row7,
           click_emb, exposure_emb, uid_emb, fc_w, fc_b):
    B, H = row0.shape
    D = click_emb.shape[1]
    V = click_emb.shape[0]
    idx = row0.astype(jnp.int32).T.reshape(H * B // _C, _C)
    idx2 = (2 * idx) % (2 * _HALF) + idx // _HALF
    table_lin = _tc_repack(click_emb.T)
    table = table_lin.reshape(2 * _HALF, D)
    out5 = _sc_gather_transposed(table, idx2, H, B)
    high_items_emb = out5.transpose(2, 4, 0, 1, 3).reshape(B, H, D)
    row1_t = row1.T
    tp_t = row6.T[-1:, :]
    fc_b2 = jnp.reshape(fc_b, (1, D))
    times_t = _tc_times_t(row1_t, tp_t, fc_w, fc_b2)
    high_times_emb = times_t.transpose(2, 0, 1)
    sep_emb = exposure_emb[0]
    return (high_items_emb, high_times_emb, sep_emb)


# speedup vs baseline: 1.3200x; 1.3200x over previous
import functools

import jax
import jax.numpy as jnp
import numpy as np
from jax import lax
from jax.experimental import pallas as pl
from jax.experimental.pallas import tpu as pltpu
from jax.experimental.pallas import tpu_sc as plsc

_C = 128
_HALF = 512000


def _tc_repack(tab_t):
    D, V = tab_t.shape
    K = 4096
    grid = _HALF // K
    nb = _HALF // K
    last = (V + K - 1) // K - 1

    def body(lo_ref, hi_ref, o_ref):
        o_ref[:, 0:D] = jnp.transpose(lo_ref[...], (1, 0))
        o_ref[:, D:2 * D] = jnp.transpose(hi_ref[...], (1, 0))

    return pl.pallas_call(
        body,
        grid=(grid,),
        in_specs=[
            pl.BlockSpec((D, K), lambda i: (0, i)),
            pl.BlockSpec((D, K), lambda i: (0, jnp.minimum(i + nb, last))),
        ],
        out_specs=pl.BlockSpec((K, 2 * D), lambda i: (i, 0)),
        out_shape=jax.ShapeDtypeStruct((_HALF, 2 * D), jnp.float32),
    )(tab_t, tab_t)


def _sc_gather(table, idx2, D):
    n_chunks = idx2.shape[0]
    N = n_chunks * _C
    info = plsc.get_sparse_core_info()
    num_cores = info.num_cores
    NW = info.num_cores * info.num_subcores
    per_w = n_chunks // NW
    mesh = plsc.VectorSubcoreMesh(core_axis_name="c", subcore_axis_name="s")

    @functools.partial(
        pl.kernel,
        mesh=mesh,
        out_type=jax.ShapeDtypeStruct((N, D), jnp.float32),
        scratch_types=[
            pltpu.VMEM((per_w, _C), jnp.int32),
            pltpu.VMEM((_C, D), jnp.float32),
            pltpu.VMEM((_C, D), jnp.float32),
            pltpu.SemaphoreType.DMA,
            pltpu.SemaphoreType.DMA,
            pltpu.SemaphoreType.DMA,
        ],
        compiler_params=pltpu.CompilerParams(use_tc_tiling_on_sc=False,
                                             needs_layout_passes=False),
    )
    def k(table_hbm, idx_hbm, out_hbm, idx_v, rows0, rows1, sem0, sem1, osem):
        wid = lax.axis_index("s") * num_cores + lax.axis_index("c")
        k0 = wid * per_w
        pltpu.sync_copy(idx_hbm.at[pl.ds(k0, per_w), :], idx_v)

        def body(t, carry):
            a = 2 * t
            g0 = pltpu.async_copy(table_hbm.at[idx_v.at[a]], rows0, sem0)
            g1 = pltpu.async_copy(table_hbm.at[idx_v.at[a + 1]], rows1, sem1)
            g0.wait()
            w0 = pltpu.async_copy(
                rows0, out_hbm.at[pl.ds((k0 + a) * _C, _C), :], osem)
            g1.wait()
            w1 = pltpu.async_copy(
                rows1, out_hbm.at[pl.ds((k0 + a + 1) * _C, _C), :], osem)
            w0.wait()
            w1.wait()
            return carry

        lax.fori_loop(0, per_w // 2, body, 0)

    return k(table, idx2)


def _tc_out_transpose(mid2, H, B, D):
    CB = B // _C
    KR = B // 2

    def body(x_ref, o_ref):
        for c in range(CB):
            xc = x_ref[pl.ds(c * (_C // 2), _C // 2), :]
            yl = jnp.transpose(xc[:, 0:D], (1, 0))
            yr = jnp.transpose(xc[:, D:2 * D], (1, 0))
            for r in range(D // 8):
                o_ref[0, r, c, :, 0:D] = yl[8 * r:8 * r + 8, :]
                o_ref[0, r, c, :, D:2 * D] = yr[8 * r:8 * r + 8, :]

    return pl.pallas_call(
        body,
        grid=(H,),
        in_specs=[pl.BlockSpec((KR, 2 * D), lambda i: (i, 0))],
        out_specs=pl.BlockSpec((1, D // 8, CB, 8, _C), lambda i: (i, 0, 0, 0, 0)),
        out_shape=jax.ShapeDtypeStruct((H, D // 8, CB, 8, _C), jnp.float32),
    )(mid2)


def _tc_times_t(row1_t, tp_t, fc_w, fc_b):
    H, B = row1_t.shape
    D = fc_w.shape[1]
    HB = 8

    def body(r1_ref, tp_ref, w_ref, b_ref, o_ref):
        t = jnp.log(tp_ref[...] - r1_ref[...] + 1.0)
        w = jnp.reshape(w_ref[...], (1, D, 1))
        bb = jnp.reshape(b_ref[...], (1, D, 1))
        o_ref[...] = t[:, None, :] * w + bb

    return pl.pallas_call(
        body,
        grid=(H // HB,),
        in_specs=[
            pl.BlockSpec((HB, B), lambda i: (i, 0)),
            pl.BlockSpec((1, B), lambda i: (0, 0)),
            pl.BlockSpec((1, D), lambda i: (0, 0)),
            pl.BlockSpec((1, D), lambda i: (0, 0)),
        ],
        out_specs=pl.BlockSpec((HB, D, B), lambda i: (i, 0, 0)),
        out_shape=jax.ShapeDtypeStruct((H, D, B), jnp.float32),
    )(row1_t, tp_t, fc_w, fc_b)


def kernel(row0, row1, row2, row3, row4, row5, row6, row7,
           click_emb, exposure_emb, uid_emb, fc_w, fc_b):
    B, H = row0.shape
    D = click_emb.shape[1]
    idx = row0.astype(jnp.int32).T.reshape(H * B // _C, _C)
    perm = jnp.asarray((np.arange(_C) % 2) * (_C // 2) + np.arange(_C) // 2,
                       dtype=jnp.int32)
    idx = idx[:, perm]
    idx2 = (2 * idx) % (2 * _HALF) + idx // _HALF
    table_lin = _tc_repack(click_emb.T)
    table = table_lin.reshape(2 * _HALF, D)
    mid = _sc_gather(table, idx2, D)
    mid2 = mid.reshape(B * H // 2, 2 * D)
    out5 = _tc_out_transpose(mid2, H, B, D)
    high_items_emb = out5.transpose(2, 4, 0, 1, 3).reshape(B, H, D)
    row1_t = row1.T
    tp_t = row6.T[-1:, :]
    fc_b2 = jnp.reshape(fc_b, (1, D))
    times_t = _tc_times_t(row1_t, tp_t, fc_w, fc_b2)
    high_times_emb = times_t.transpose(2, 0, 1)
    sep_emb = exposure_emb[0]
    return (high_items_emb, high_times_emb, sep_emb)


# speedup vs baseline: 2.4331x; 1.8432x over previous
import functools

import jax
import jax.numpy as jnp
import numpy as np
from jax import lax
from jax.experimental import pallas as pl
from jax.experimental.pallas import tpu as pltpu
from jax.experimental.pallas import tpu_sc as plsc

_C = 128
_HALF = 512000


def _tc_repack(tab_t):
    D, V = tab_t.shape
    K = 4096
    grid = _HALF // K
    nb = _HALF // K
    last = (V + K - 1) // K - 1

    def body(lo_ref, hi_ref, o_ref):
        o_ref[:, 0:D] = jnp.transpose(lo_ref[...], (1, 0))
        o_ref[:, D:2 * D] = jnp.transpose(hi_ref[...], (1, 0))

    return pl.pallas_call(
        body,
        grid=(grid,),
        in_specs=[
            pl.BlockSpec((D, K), lambda i: (0, i)),
            pl.BlockSpec((D, K), lambda i: (0, jnp.minimum(i + nb, last))),
        ],
        out_specs=pl.BlockSpec((K, 2 * D), lambda i: (i, 0)),
        out_shape=jax.ShapeDtypeStruct((_HALF, 2 * D), jnp.float32),
    )(tab_t, tab_t)


def _sc_gather(table, idx2, D):
    n_chunks = idx2.shape[0]
    N = n_chunks * _C
    info = plsc.get_sparse_core_info()
    num_cores = info.num_cores
    NW = info.num_cores * info.num_subcores
    per_w = n_chunks // NW
    mesh = plsc.VectorSubcoreMesh(core_axis_name="c", subcore_axis_name="s")

    @functools.partial(
        pl.kernel,
        mesh=mesh,
        out_type=jax.ShapeDtypeStruct((N, D), jnp.float32),
        scratch_types=[
            pltpu.VMEM((per_w, _C), jnp.int32),
            pltpu.VMEM((_C, D), jnp.float32),
            pltpu.VMEM((_C, D), jnp.float32),
            pltpu.SemaphoreType.DMA,
            pltpu.SemaphoreType.DMA,
            pltpu.SemaphoreType.DMA,
        ],
        compiler_params=pltpu.CompilerParams(use_tc_tiling_on_sc=False,
                                             needs_layout_passes=False),
    )
    def k(table_hbm, idx_hbm, out_hbm, idx_v, rows0, rows1, sem0, sem1, osem):
        wid = lax.axis_index("s") * num_cores + lax.axis_index("c")
        k0 = wid * per_w
        pltpu.sync_copy(idx_hbm.at[pl.ds(k0, per_w), :], idx_v)

        def body(t, carry):
            a = 2 * t
            g0 = pltpu.async_copy(table_hbm.at[idx_v.at[a]], rows0, sem0)
            g1 = pltpu.async_copy(table_hbm.at[idx_v.at[a + 1]], rows1, sem1)
            g0.wait()
            w0 = pltpu.async_copy(
                rows0, out_hbm.at[pl.ds((k0 + a) * _C, _C), :], osem)
            g1.wait()
            w1 = pltpu.async_copy(
                rows1, out_hbm.at[pl.ds((k0 + a + 1) * _C, _C), :], osem)
            w0.wait()
            w1.wait()
            return carry

        lax.fori_loop(0, per_w // 2, body, 0)

    return k(table, idx2)


def _tc_out_transpose(mid2, H, B, D):
    CB = B // _C
    KR = B // 2

    def body(x_ref, o_ref):
        y = jnp.transpose(x_ref[...], (1, 0))
        for c in range(CB):
            p, cc = (0, c) if c < CB // 2 else (1, c - CB // 2)
            for r in range(D // 8):
                o_ref[0, r, c, :, :] = (
                    y[p * D + 8 * r:p * D + 8 * r + 8,
                      cc * _C:cc * _C + _C])

    return pl.pallas_call(
        body,
        grid=(H,),
        in_specs=[pl.BlockSpec((KR, 2 * D), lambda i: (i, 0))],
        out_specs=pl.BlockSpec((1, D // 8, CB, 8, _C), lambda i: (i, 0, 0, 0, 0)),
        out_shape=jax.ShapeDtypeStruct((H, D // 8, CB, 8, _C), jnp.float32),
    )(mid2)


def _tc_times_t(row1_t, tp_t, fc_w, fc_b):
    H, B = row1_t.shape
    D = fc_w.shape[1]
    HB = 8

    def body(r1_ref, tp_ref, w_ref, b_ref, o_ref):
        t = jnp.log(tp_ref[...] - r1_ref[...] + 1.0)
        w = jnp.reshape(w_ref[...], (1, D, 1))
        bb = jnp.reshape(b_ref[...], (1, D, 1))
        o_ref[...] = t[:, None, :] * w + bb

    return pl.pallas_call(
        body,
        grid=(H // HB,),
        in_specs=[
            pl.BlockSpec((HB, B), lambda i: (i, 0)),
            pl.BlockSpec((1, B), lambda i: (0, 0)),
            pl.BlockSpec((1, D), lambda i: (0, 0)),
            pl.BlockSpec((1, D), lambda i: (0, 0)),
        ],
        out_specs=pl.BlockSpec((HB, D, B), lambda i: (i, 0, 0)),
        out_shape=jax.ShapeDtypeStruct((H, D, B), jnp.float32),
    )(row1_t, tp_t, fc_w, fc_b)


def kernel(row0, row1, row2, row3, row4, row5, row6, row7,
           click_emb, exposure_emb, uid_emb, fc_w, fc_b):
    B, H = row0.shape
    D = click_emb.shape[1]
    pg = jnp.asarray((np.arange(B) % 2) * (B // 2) + np.arange(B) // 2,
                     dtype=jnp.int32)
    idx = row0.astype(jnp.int32).T[:, pg].reshape(H * B // _C, _C)
    idx2 = (2 * idx) % (2 * _HALF) + idx // _HALF
    table_lin = _tc_repack(click_emb.T)
    table = table_lin.reshape(2 * _HALF, D)
    mid = _sc_gather(table, idx2, D)
    mid2 = mid.reshape(B * H // 2, 2 * D)
    out5 = _tc_out_transpose(mid2, H, B, D)
    high_items_emb = out5.transpose(2, 4, 0, 1, 3).reshape(B, H, D)
    row1_t = row1.T
    tp_t = row6.T[-1:, :]
    fc_b2 = jnp.reshape(fc_b, (1, D))
    times_t = _tc_times_t(row1_t, tp_t, fc_w, fc_b2)
    high_times_emb = times_t.transpose(2, 0, 1)
    sep_emb = exposure_emb[0]
    return (high_items_emb, high_times_emb, sep_emb)


# speedup vs baseline: 2.4431x; 1.0041x over previous
import functools

import jax
import jax.numpy as jnp
import numpy as np
from jax import lax
from jax.experimental import pallas as pl
from jax.experimental.pallas import tpu as pltpu
from jax.experimental.pallas import tpu_sc as plsc

_C = 128
_HALF = 512000


def _tc_repack(tab_t):
    D, V = tab_t.shape
    K = 4096
    grid = _HALF // K
    nb = _HALF // K
    last = (V + K - 1) // K - 1

    def body(lo_ref, hi_ref, o_ref):
        o_ref[:, 0:D] = jnp.transpose(lo_ref[...], (1, 0))
        o_ref[:, D:2 * D] = jnp.transpose(hi_ref[...], (1, 0))

    return pl.pallas_call(
        body,
        grid=(grid,),
        in_specs=[
            pl.BlockSpec((D, K), lambda i: (0, i)),
            pl.BlockSpec((D, K), lambda i: (0, jnp.minimum(i + nb, last))),
        ],
        out_specs=pl.BlockSpec((K, 2 * D), lambda i: (i, 0)),
        out_shape=jax.ShapeDtypeStruct((_HALF, 2 * D), jnp.float32),
    )(tab_t, tab_t)


def _sc_gather(table, idx2, D):
    n_chunks = idx2.shape[0]
    N = n_chunks * _C
    info = plsc.get_sparse_core_info()
    num_cores = info.num_cores
    NW = info.num_cores * info.num_subcores
    per_w = n_chunks // NW
    mesh = plsc.VectorSubcoreMesh(core_axis_name="c", subcore_axis_name="s")

    @functools.partial(
        pl.kernel,
        mesh=mesh,
        out_type=jax.ShapeDtypeStruct((N, D), jnp.float32),
        scratch_types=[
            pltpu.VMEM((per_w, _C), jnp.int32),
            pltpu.VMEM((4, _C, D), jnp.float32),
            pltpu.SemaphoreType.DMA,
            pltpu.SemaphoreType.DMA,
            pltpu.SemaphoreType.DMA,
            pltpu.SemaphoreType.DMA,
            pltpu.SemaphoreType.DMA,
        ],
        compiler_params=pltpu.CompilerParams(use_tc_tiling_on_sc=False,
                                             needs_layout_passes=False),
    )
    def k(table_hbm, idx_hbm, out_hbm, idx_v, rows, s0, s1, s2, s3, osem):
        wid = lax.axis_index("s") * num_cores + lax.axis_index("c")
        k0 = wid * per_w
        pltpu.sync_copy(idx_hbm.at[pl.ds(k0, per_w), :], idx_v)
        sems = [s0, s1, s2, s3]

        def body(t, carry):
            a = 4 * t
            gs = [pltpu.async_copy(table_hbm.at[idx_v.at[a + i]],
                                   rows.at[i], sems[i]) for i in range(4)]
            ws = []
            for i in range(4):
                gs[i].wait()
                ws.append(pltpu.async_copy(
                    rows.at[i],
                    out_hbm.at[pl.ds((k0 + a + i) * _C, _C), :], osem))
            for w in ws:
                w.wait()
            return carry

        lax.fori_loop(0, per_w // 4, body, 0)

    return k(table, idx2)


def _tc_out_transpose(mid2, H, B, D):
    CB = B // _C
    KR = B // 2

    def body(x_ref, o_ref):
        y = jnp.transpose(x_ref[...], (1, 0))
        for c in range(CB):
            p, cc = (0, c) if c < CB // 2 else (1, c - CB // 2)
            for r in range(D // 8):
                o_ref[0, r, c, :, :] = (
                    y[p * D + 8 * r:p * D + 8 * r + 8,
                      cc * _C:cc * _C + _C])

    return pl.pallas_call(
        body,
        grid=(H,),
        in_specs=[pl.BlockSpec((KR, 2 * D), lambda i: (i, 0))],
        out_specs=pl.BlockSpec((1, D // 8, CB, 8, _C), lambda i: (i, 0, 0, 0, 0)),
        out_shape=jax.ShapeDtypeStruct((H, D // 8, CB, 8, _C), jnp.float32),
    )(mid2)


def _tc_times_t(row1_t, tp_t, fc_w, fc_b):
    H, B = row1_t.shape
    D = fc_w.shape[1]
    HB = 8

    def body(r1_ref, tp_ref, w_ref, b_ref, o_ref):
        t = jnp.log(tp_ref[...] - r1_ref[...] + 1.0)
        w = jnp.reshape(w_ref[...], (1, D, 1))
        bb = jnp.reshape(b_ref[...], (1, D, 1))
        o_ref[...] = t[:, None, :] * w + bb

    return pl.pallas_call(
        body,
        grid=(H // HB,),
        in_specs=[
            pl.BlockSpec((HB, B), lambda i: (i, 0)),
            pl.BlockSpec((1, B), lambda i: (0, 0)),
            pl.BlockSpec((1, D), lambda i: (0, 0)),
            pl.BlockSpec((1, D), lambda i: (0, 0)),
        ],
        out_specs=pl.BlockSpec((HB, D, B), lambda i: (i, 0, 0)),
        out_shape=jax.ShapeDtypeStruct((H, D, B), jnp.float32),
    )(row1_t, tp_t, fc_w, fc_b)


def kernel(row0, row1, row2, row3, row4, row5, row6, row7,
           click_emb, exposure_emb, uid_emb, fc_w, fc_b):
    B, H = row0.shape
    D = click_emb.shape[1]
    pg = jnp.asarray((np.arange(B) % 2) * (B // 2) + np.arange(B) // 2,
                     dtype=jnp.int32)
    idx = row0.astype(jnp.int32).T[:, pg].reshape(H * B // _C, _C)
    idx2 = (2 * idx) % (2 * _HALF) + idx // _HALF
    table_lin = _tc_repack(click_emb.T)
    table = table_lin.reshape(2 * _HALF, D)
    mid = _sc_gather(table, idx2, D)
    mid2 = mid.reshape(B * H // 2, 2 * D)
    out5 = _tc_out_transpose(mid2, H, B, D)
    high_items_emb = out5.transpose(2, 4, 0, 1, 3).reshape(B, H, D)
    row1_t = row1.T
    tp_t = row6.T[-1:, :]
    fc_b2 = jnp.reshape(fc_b, (1, D))
    times_t = _tc_times_t(row1_t, tp_t, fc_w, fc_b2)
    high_times_emb = times_t.transpose(2, 0, 1)
    sep_emb = exposure_emb[0]
    return (high_items_emb, high_times_emb, sep_emb)
